# baseline (speedup 1.0000x reference)
import functools
import math

import jax
import jax.numpy as jnp
from jax import lax
from jax.experimental import pallas as pl
from jax.experimental.pallas import tpu as pltpu
from jax.experimental.pallas import tpu_sc as plsc

_S = 64.0
_M = 0.5
_COS_M = math.cos(_M)
_SIN_M = math.sin(_M)
_LOG2E = math.log2(math.e)


def _sumexp_kernel(xt_ref, out_ref, acc_sum, *, bn, n):
    c = pl.program_id(0)
    nc = pl.num_programs(0)
    k = _S * _LOG2E

    @pl.when(c == 0)
    def _():
        acc_sum[...] = jnp.zeros_like(acc_sum)

    x = xt_ref[...]

    @pl.when(c < nc - 1)
    def _():
        acc_sum[...] += jnp.sum(jnp.exp2(x * k - k), axis=0, keepdims=True)

    @pl.when(c == nc - 1)
    def _():
        rows_local = jax.lax.broadcasted_iota(jnp.int32, x.shape, 0)
        e = jnp.exp2(jnp.where(rows_local < n - c * bn, x * k - k, -1e30))
        acc_sum[...] += jnp.sum(e, axis=0, keepdims=True)
        out_ref[...] = acc_sum[...]


def _finish_kernel(sums_ref, rows16_ref, out_ref):
    k = _S * _LOG2E
    rows16 = rows16_ref[...]
    lane = jax.lax.broadcasted_iota(jnp.int32, rows16.shape, 1)
    row = jax.lax.broadcasted_iota(jnp.int32, rows16.shape, 0)
    sel = lane == (row % 128)
    p = jnp.sum(jnp.where(sel, rows16, 0.0), axis=1, keepdims=True)
    s = sums_ref[...]
    m = p * _COS_M - jnp.sqrt(jnp.maximum(1.0 - p * p, 0.0)) * _SIN_M
    s2 = s - jnp.exp2(p * k - k) + jnp.exp2(m * k - k)
    loss = (_S + jnp.log(s2)) - m * _S
    out_ref[...] = jnp.sum(loss, axis=(0, 1), keepdims=True) / loss.shape[0]


def _make_sc_gather(b):
    info = plsc.get_sparse_core_info()
    nw = info.num_cores * info.num_subcores
    b_per_w = b // nw
    mesh = plsc.VectorSubcoreMesh(core_axis_name="c", subcore_axis_name="s")

    @functools.partial(
        pl.kernel,
        mesh=mesh,
        out_type=jax.ShapeDtypeStruct((b, 128), jnp.float32),
        scratch_types=[
            pltpu.VMEM((b_per_w,), jnp.int32),
            pltpu.VMEM((b_per_w, 128), jnp.float32),
            pltpu.SemaphoreType.DMA,
        ],
    )
    def sc_gather(table_hbm, idx_hbm, out_hbm, idx_v, rows_v, sem):
        wid = lax.axis_index("s") * info.num_cores + lax.axis_index("c")
        base = wid * b_per_w
        pltpu.sync_copy(idx_hbm.at[pl.ds(base, b_per_w)], idx_v)
        pltpu.async_copy(table_hbm.at[idx_v], rows_v, sem).wait()
        pltpu.sync_copy(rows_v, out_hbm.at[pl.ds(base, b_per_w)])

    return sc_gather


def kernel(cosine, labels):
    if labels.ndim == 2:
        labels = labels.squeeze(1)
    b, n = cosine.shape
    xt = cosine.T
    labels32 = labels.astype(jnp.int32)

    table = xt.reshape(n * b // 128, 128)
    idx = labels32 * (b // 128) + jnp.arange(b, dtype=jnp.int32) // 128
    rows16 = _make_sc_gather(b)(table, idx)

    bn = 2048
    grid = (pl.cdiv(n, bn),)
    sums = pl.pallas_call(
        functools.partial(_sumexp_kernel, bn=bn, n=n),
        grid=grid,
        in_specs=[pl.BlockSpec((bn, b), lambda c: (c, 0))],
        out_specs=pl.BlockSpec((1, b), lambda c: (0, 0)),
        out_shape=jax.ShapeDtypeStruct((1, b), jnp.float32),
        scratch_shapes=[pltpu.VMEM((1, b), jnp.float32)],
    )(xt)

    loss = pl.pallas_call(
        _finish_kernel,
        out_shape=jax.ShapeDtypeStruct((1, 1), jnp.float32),
    )(sums.reshape(b, 1), rows16)
    return loss.reshape(())


# speedup vs baseline: 3.3031x; 3.3031x over previous
import functools
import math

import jax
import jax.numpy as jnp
from jax import lax
from jax.experimental import pallas as pl
from jax.experimental.pallas import tpu as pltpu
from jax.experimental.pallas import tpu_sc as plsc

_S = 64.0
_M = 0.5
_COS_M = math.cos(_M)
_SIN_M = math.sin(_M)
_LOG2E = math.log2(math.e)


def _sumexp_kernel(xt_ref, out_ref, acc_sum, *, bn, n):
    c = pl.program_id(0)
    nc = pl.num_programs(0)
    k = _S * _LOG2E

    @pl.when(c == 0)
    def _():
        acc_sum[...] = jnp.zeros_like(acc_sum)

    x = xt_ref[...]

    @pl.when(c < nc - 1)
    def _():
        acc_sum[...] += jnp.sum(jnp.exp2(x * k - k), axis=0, keepdims=True)

    @pl.when(c == nc - 1)
    def _():
        rows_local = jax.lax.broadcasted_iota(jnp.int32, x.shape, 0)
        e = jnp.exp2(jnp.where(rows_local < n - c * bn, x * k - k, -1e30))
        acc_sum[...] += jnp.sum(e, axis=0, keepdims=True)
        out_ref[...] = acc_sum[...]


def _finish_kernel(sums_ref, rows_ref, out_ref):
    k = _S * _LOG2E
    rows = rows_ref[...]
    r_i = jax.lax.broadcasted_iota(jnp.int32, rows.shape, 0)
    c_i = jax.lax.broadcasted_iota(jnp.int32, rows.shape, 1)
    p = jnp.sum(jnp.where(r_i == c_i, rows, 0.0), axis=1, keepdims=True)
    s = sums_ref[...]
    m = p * _COS_M - jnp.sqrt(jnp.maximum(1.0 - p * p, 0.0)) * _SIN_M
    s2 = s - jnp.exp2(p * k - k) + jnp.exp2(m * k - k)
    loss = (_S + jnp.log(s2)) - m * _S
    out_ref[...] = jnp.sum(loss, axis=(0, 1), keepdims=True) / loss.shape[0]


def _make_sc_gather(b):
    info = plsc.get_sparse_core_info()
    nw = info.num_cores * info.num_subcores
    b_per_w = b // nw
    mesh = plsc.VectorSubcoreMesh(core_axis_name="c", subcore_axis_name="s")

    @functools.partial(
        pl.kernel,
        mesh=mesh,
        out_type=jax.ShapeDtypeStruct((b, b), jnp.float32),
        scratch_types=[
            pltpu.VMEM((b_per_w,), jnp.int32),
            pltpu.VMEM((b_per_w, b), jnp.float32),
            pltpu.SemaphoreType.DMA,
        ],
    )
    def sc_gather(table_hbm, idx_hbm, out_hbm, idx_v, rows_v, sem):
        wid = lax.axis_index("s") * info.num_cores + lax.axis_index("c")
        base = wid * b_per_w
        pltpu.sync_copy(idx_hbm.at[pl.ds(base, b_per_w)], idx_v)
        pltpu.async_copy(table_hbm.at[idx_v], rows_v, sem).wait()
        pltpu.sync_copy(rows_v, out_hbm.at[pl.ds(base, b_per_w)])

    return sc_gather


def kernel(cosine, labels):
    if labels.ndim == 2:
        labels = labels.squeeze(1)
    b, n = cosine.shape
    xt = cosine.T
    labels32 = labels.astype(jnp.int32)

    rows = _make_sc_gather(b)(xt, labels32)

    bn = 2048
    grid = (pl.cdiv(n, bn),)
    sums = pl.pallas_call(
        functools.partial(_sumexp_kernel, bn=bn, n=n),
        grid=grid,
        in_specs=[pl.BlockSpec((bn, b), lambda c: (c, 0))],
        out_specs=pl.BlockSpec((1, b), lambda c: (0, 0)),
        out_shape=jax.ShapeDtypeStruct((1, b), jnp.float32),
        scratch_shapes=[pltpu.VMEM((1, b), jnp.float32)],
    )(xt)

    loss = pl.pallas_call(
        _finish_kernel,
        out_shape=jax.ShapeDtypeStruct((1, 1), jnp.float32),
    )(sums.reshape(b, 1), rows)
    return loss.reshape(())


# speedup vs baseline: 3.4486x; 1.0440x over previous
import functools
import math

import jax
import jax.numpy as jnp
from jax import lax
from jax.experimental import pallas as pl
from jax.experimental.pallas import tpu as pltpu
from jax.experimental.pallas import tpu_sc as plsc

_S = 64.0
_M = 0.5
_COS_M = math.cos(_M)
_SIN_M = math.sin(_M)
_LOG2E = math.log2(math.e)


def _sumexp_kernel(xt_ref, out_ref, acc_sum, *, bn, n):
    c = pl.program_id(0)
    nc = pl.num_programs(0)
    k = _S * _LOG2E

    @pl.when(c == 0)
    def _():
        acc_sum[...] = jnp.zeros_like(acc_sum)

    x = xt_ref[...]

    @pl.when(c < nc - 1)
    def _():
        acc_sum[...] += jnp.sum(jnp.exp2(x * k - k), axis=0, keepdims=True)

    @pl.when(c == nc - 1)
    def _():
        rows_local = jax.lax.broadcasted_iota(jnp.int32, x.shape, 0)
        e = jnp.exp2(jnp.where(rows_local < n - c * bn, x * k - k, -1e30))
        acc_sum[...] += jnp.sum(e, axis=0, keepdims=True)
        out_ref[...] = acc_sum[...]


def _finish_kernel(sums_ref, rows_ref, out_ref):
    k = _S * _LOG2E
    rows = rows_ref[...]
    r_i = jax.lax.broadcasted_iota(jnp.int32, rows.shape, 0)
    c_i = jax.lax.broadcasted_iota(jnp.int32, rows.shape, 1)
    p = jnp.sum(jnp.where(r_i == c_i, rows, 0.0), axis=1, keepdims=True)
    s = sums_ref[...]
    m = p * _COS_M - jnp.sqrt(jnp.maximum(1.0 - p * p, 0.0)) * _SIN_M
    s2 = s - jnp.exp2(p * k - k) + jnp.exp2(m * k - k)
    loss = (_S + jnp.log(s2)) - m * _S
    out_ref[...] = jnp.sum(loss, axis=(0, 1), keepdims=True) / loss.shape[0]


def _make_sc_gather(b):
    info = plsc.get_sparse_core_info()
    nw = info.num_cores * info.num_subcores
    b_per_w = b // nw
    mesh = plsc.VectorSubcoreMesh(core_axis_name="c", subcore_axis_name="s")

    @functools.partial(
        pl.kernel,
        mesh=mesh,
        out_type=jax.ShapeDtypeStruct((b, b), jnp.float32),
        scratch_types=[
            pltpu.VMEM((b_per_w,), jnp.int32),
            pltpu.VMEM((b_per_w, b), jnp.float32),
            pltpu.SemaphoreType.DMA,
        ],
    )
    def sc_gather(table_hbm, idx_hbm, out_hbm, idx_v, rows_v, sem):
        wid = lax.axis_index("s") * info.num_cores + lax.axis_index("c")
        base = wid * b_per_w
        pltpu.sync_copy(idx_hbm.at[pl.ds(base, b_per_w)], idx_v)
        pltpu.async_copy(table_hbm.at[idx_v], rows_v, sem).wait()
        pltpu.sync_copy(rows_v, out_hbm.at[pl.ds(base, b_per_w)])

    return sc_gather


def kernel(cosine, labels):
    if labels.ndim == 2:
        labels = labels.squeeze(1)
    b, n = cosine.shape
    xt = cosine.T
    labels32 = labels.astype(jnp.int32)

    rows = _make_sc_gather(b)(xt, labels32)

    bn = 3072
    grid = (pl.cdiv(n, bn),)
    sums = pl.pallas_call(
        functools.partial(_sumexp_kernel, bn=bn, n=n),
        grid=grid,
        in_specs=[pl.BlockSpec((bn, b), lambda c: (c, 0))],
        out_specs=pl.BlockSpec((1, b), lambda c: (0, 0)),
        out_shape=jax.ShapeDtypeStruct((1, b), jnp.float32),
        scratch_shapes=[pltpu.VMEM((1, b), jnp.float32)],
    )(xt)

    loss = pl.pallas_call(
        _finish_kernel,
        out_shape=jax.ShapeDtypeStruct((1, 1), jnp.float32),
    )(sums.reshape(b, 1), rows)
    return loss.reshape(())
